# baseline (speedup 1.0000x reference)
import functools

import jax
import jax.numpy as jnp
from jax import lax
from jax.experimental import pallas as pl
from jax.experimental.pallas import tpu as pltpu
from jax.experimental.pallas import tpu_sc as plsc

N_NODES = 10000
N_EDGES = 320000
D_IN = 128
D_EDGE = 16
D_OUT = 128

NC = 2
NS = 16
NW = NC * NS
CH = 128
R = 10112
RT = R // NS
ITERS = 79
EPW = ITERS * CH
EP = NW * EPW
DEA = 32

_MESH = dict(core_axis_name="c", subcore_axis_name="s")


def _sc_gather_nodes(nfeats, src3, dst3):
  @functools.partial(
      pl.kernel,
      mesh=plsc.VectorSubcoreMesh(**_MESH),
      out_type=jax.ShapeDtypeStruct((NC, R, D_IN), jnp.float32),
      scratch_types=[
          pltpu.VMEM((CH, D_IN), jnp.float32),
          pltpu.VMEM((ITERS, CH), jnp.int32),
          pltpu.VMEM((ITERS, CH), jnp.int32),
          pltpu.VMEM_SHARED((R, D_IN), jnp.float32),
          pltpu.SemaphoreType.DMA,
      ],
  )
  def k(nf_hbm, src_hbm, dst_hbm, g_out, buf0, sidx_v, didx_v, g_sh, sga):
    c = lax.axis_index("c")
    s = lax.axis_index("s")
    wid = c * NS + s
    tb = s * RT

    zeros16 = jnp.zeros((16,), jnp.float32)

    def zrow(i, _):
      for l in range(D_IN // 16):
        buf0[i, pl.ds(l * 16, 16)] = zeros16
      return 0
    lax.fori_loop(0, CH, zrow, 0)

    def zcp(kk, _):
      pltpu.sync_copy(buf0, g_sh.at[pl.ds(tb + kk * CH, CH)])
      return 0
    nfull = RT // CH
    rem = RT - nfull * CH
    lax.fori_loop(0, nfull, zcp, 0)
    pltpu.sync_copy(buf0.at[pl.ds(0, rem)], g_sh.at[pl.ds(tb + nfull * CH, rem)])

    pltpu.sync_copy(src_hbm.at[wid], sidx_v)
    pltpu.sync_copy(dst_hbm.at[wid], didx_v)
    plsc.subcore_barrier()

    def chunk(j, _):
      pltpu.async_copy(nf_hbm.at[sidx_v.at[j]], buf0, sga).wait()
      pltpu.sync_copy(buf0, g_sh.at[didx_v.at[j]], add=True)
      return 0
    lax.fori_loop(0, ITERS, chunk, 0)

    plsc.subcore_barrier()
    pltpu.sync_copy(g_sh.at[pl.ds(tb, RT)], g_out.at[c, pl.ds(tb, RT)])

  return k(nfeats, src3, dst3)


def _sc_gather_edges(efa, dst3):
  @functools.partial(
      pl.kernel,
      mesh=plsc.VectorSubcoreMesh(**_MESH),
      out_type=jax.ShapeDtypeStruct((NC, R, D_IN), jnp.float32),
      scratch_types=[
          pltpu.VMEM((CH, DEA), jnp.float32),
          pltpu.VMEM((CH, D_IN), jnp.float32),
          pltpu.VMEM((ITERS, CH), jnp.int32),
          pltpu.VMEM_SHARED((R, D_IN), jnp.float32),
          pltpu.SemaphoreType.DMA,
      ],
  )
  def k(efa_hbm, dst_hbm, ec_out, ebuf0, wide0, didx_v, ec_sh, sla):
    c = lax.axis_index("c")
    s = lax.axis_index("s")
    wid = c * NS + s
    tb = s * RT

    zeros16 = jnp.zeros((16,), jnp.float32)
    def zrow(i, _):
      for l in range(D_IN // 16):
        wide0[i, pl.ds(l * 16, 16)] = zeros16
      return 0
    lax.fori_loop(0, CH, zrow, 0)

    def zcp(kk, _):
      pltpu.sync_copy(wide0, ec_sh.at[pl.ds(tb + kk * CH, CH)])
      return 0
    nfull = RT // CH
    rem = RT - nfull * CH
    lax.fori_loop(0, nfull, zcp, 0)
    pltpu.sync_copy(wide0.at[pl.ds(0, rem)], ec_sh.at[pl.ds(tb + nfull * CH, rem)])

    pltpu.sync_copy(dst_hbm.at[wid], didx_v)
    plsc.subcore_barrier()

    ebase = wid * EPW
    def chunk(j, _):
      pltpu.async_copy(
          efa_hbm.at[pl.ds(ebase + j * CH, CH)], ebuf0, sla).wait()
      def wrow(i, _):
        wide0[i, pl.ds(0, 16)] = ebuf0[i, pl.ds(0, 16)]
        wide0[i, pl.ds(16, 16)] = ebuf0[i, pl.ds(16, 16)]
        return 0
      lax.fori_loop(0, CH, wrow, 0)
      pltpu.sync_copy(wide0, ec_sh.at[didx_v.at[j]], add=True)
      return 0
    lax.fori_loop(0, ITERS, chunk, 0)

    plsc.subcore_barrier()
    pltpu.sync_copy(ec_sh.at[pl.ds(tb, RT)], ec_out.at[c, pl.ds(tb, RT)])

  return k(efa, dst3)


def _tc_finish(g2, ec2, nfeats, wmsg_t, wapply_t, b_msg, b_apply):
  B = 2000
  grid = (N_NODES // B,)

  def body(g_ref, ec_ref, nf_ref, wm_ref, wa_ref, bm_ref, ba_ref, out_ref):
    g = g_ref[0] + g_ref[1]
    ec = ec_ref[0] + ec_ref[1]
    e = ec[:, :D_EDGE]
    cnt = ec[:, D_EDGE:D_EDGE + 1]
    summ = (
        jnp.dot(g, wm_ref[:D_IN], preferred_element_type=jnp.float32)
        + jnp.dot(e, wm_ref[D_IN:D_IN + D_EDGE], preferred_element_type=jnp.float32)
        + cnt * bm_ref[...]
    )
    h_neigh = summ / jnp.maximum(cnt, 1.0)
    ap = (
        jnp.dot(nf_ref[...], wa_ref[:D_IN], preferred_element_type=jnp.float32)
        + jnp.dot(h_neigh, wa_ref[D_IN:], preferred_element_type=jnp.float32)
        + ba_ref[...]
    )
    out_ref[...] = jnp.maximum(ap, 0.0)

  return pl.pallas_call(
      body,
      grid=grid,
      in_specs=[
          pl.BlockSpec((NC, B, D_IN), lambda i: (0, i, 0)),
          pl.BlockSpec((NC, B, D_IN), lambda i: (0, i, 0)),
          pl.BlockSpec((B, D_IN), lambda i: (i, 0)),
          pl.BlockSpec((D_IN + D_EDGE, D_OUT), lambda i: (0, 0)),
          pl.BlockSpec((D_IN + D_OUT, D_OUT), lambda i: (0, 0)),
          pl.BlockSpec((1, D_OUT), lambda i: (0, 0)),
          pl.BlockSpec((1, D_OUT), lambda i: (0, 0)),
      ],
      out_specs=pl.BlockSpec((B, D_OUT), lambda i: (i, 0)),
      out_shape=jax.ShapeDtypeStruct((N_NODES, D_OUT), jnp.float32),
  )(g2, ec2, nfeats, wmsg_t, wapply_t, b_msg, b_apply)


@jax.jit
def kernel(nfeats, efeats, edge_index, W_msg, b_msg, W_apply, b_apply):
  npad = EP - N_EDGES
  src = jnp.concatenate(
      [edge_index[0].astype(jnp.int32), jnp.zeros((npad,), jnp.int32)])
  dst = jnp.concatenate(
      [edge_index[1].astype(jnp.int32),
       jnp.full((npad,), R - 1, jnp.int32)])
  src = src.reshape(NW, ITERS, CH)
  dst = dst.reshape(NW, ITERS, CH)
  nfeats = nfeats.astype(jnp.float32)
  efa = jnp.concatenate(
      [jnp.concatenate(
          [efeats.astype(jnp.float32),
           jnp.ones((N_EDGES, 1), jnp.float32),
           jnp.zeros((N_EDGES, DEA - D_EDGE - 1), jnp.float32)], axis=1),
       jnp.zeros((npad, DEA), jnp.float32)])

  g2 = _sc_gather_nodes(nfeats, src, dst)
  ec2 = _sc_gather_edges(efa, dst)

  return _tc_finish(
      g2[:, :N_NODES], ec2[:, :N_NODES], nfeats,
      W_msg.T.astype(jnp.float32), W_apply.T.astype(jnp.float32),
      b_msg.reshape(1, D_OUT), b_apply.reshape(1, D_OUT))


# speedup vs baseline: 1.3079x; 1.3079x over previous
import functools

import jax
import jax.numpy as jnp
from jax import lax
from jax.experimental import pallas as pl
from jax.experimental.pallas import tpu as pltpu
from jax.experimental.pallas import tpu_sc as plsc

N_NODES = 10000
N_EDGES = 320000
D_IN = 128
D_EDGE = 16
D_OUT = 128

NC = 2
NS = 16
NW = NC * NS
CH = 128
R = 10112
RT = R // NS
ITERS = 79
EPW = ITERS * CH
EP = NW * EPW
DEA = 32

_MESH = dict(core_axis_name="c", subcore_axis_name="s")


def _sc_gather_nodes(nfeats, src3, dst3):
  @functools.partial(
      pl.kernel,
      mesh=plsc.VectorSubcoreMesh(**_MESH),
      out_type=jax.ShapeDtypeStruct((NC, R, D_IN), jnp.float32),
      scratch_types=[
          pltpu.VMEM((CH, D_IN), jnp.float32),
          pltpu.VMEM((ITERS, CH), jnp.int32),
          pltpu.VMEM((ITERS, CH), jnp.int32),
          pltpu.VMEM_SHARED((R, D_IN), jnp.float32),
          pltpu.SemaphoreType.DMA,
      ],
  )
  def k(nf_hbm, src_hbm, dst_hbm, g_out, buf0, sidx_v, didx_v, g_sh, sga):
    c = lax.axis_index("c")
    s = lax.axis_index("s")
    wid = c * NS + s
    tb = s * RT

    zeros16 = jnp.zeros((16,), jnp.float32)

    def zrow(i, _):
      for l in range(D_IN // 16):
        buf0[i, pl.ds(l * 16, 16)] = zeros16
      return 0
    lax.fori_loop(0, CH, zrow, 0)

    def zcp(kk, _):
      pltpu.sync_copy(buf0, g_sh.at[pl.ds(tb + kk * CH, CH)])
      return 0
    nfull = RT // CH
    rem = RT - nfull * CH
    lax.fori_loop(0, nfull, zcp, 0)
    pltpu.sync_copy(buf0.at[pl.ds(0, rem)], g_sh.at[pl.ds(tb + nfull * CH, rem)])

    pltpu.sync_copy(src_hbm.at[wid], sidx_v)
    pltpu.sync_copy(dst_hbm.at[wid], didx_v)
    plsc.subcore_barrier()

    def chunk(j, _):
      pltpu.async_copy(nf_hbm.at[sidx_v.at[j]], buf0, sga).wait()
      pltpu.sync_copy(buf0, g_sh.at[didx_v.at[j]], add=True)
      return 0
    lax.fori_loop(0, ITERS, chunk, 0)

    plsc.subcore_barrier()
    pltpu.sync_copy(g_sh.at[pl.ds(tb, RT)], g_out.at[c, pl.ds(tb, RT)])

  return k(nfeats, src3, dst3)


def _sc_gather_edges(efa, dst3):
  @functools.partial(
      pl.kernel,
      mesh=plsc.VectorSubcoreMesh(**_MESH),
      out_type=jax.ShapeDtypeStruct((NC, R, D_IN), jnp.float32),
      scratch_types=[
          pltpu.VMEM((CH, DEA), jnp.float32),
          pltpu.VMEM((CH, D_IN), jnp.float32),
          pltpu.VMEM((ITERS, CH), jnp.int32),
          pltpu.VMEM_SHARED((R, D_IN), jnp.float32),
          pltpu.SemaphoreType.DMA,
      ],
  )
  def k(efa_hbm, dst_hbm, ec_out, ebuf0, wide0, didx_v, ec_sh, sla):
    c = lax.axis_index("c")
    s = lax.axis_index("s")
    wid = c * NS + s
    tb = s * RT

    zeros16 = jnp.zeros((16,), jnp.float32)
    def zrow(i, _):
      for l in range(D_IN // 16):
        wide0[i, pl.ds(l * 16, 16)] = zeros16
      return 0
    lax.fori_loop(0, CH, zrow, 0)

    def zcp(kk, _):
      pltpu.sync_copy(wide0, ec_sh.at[pl.ds(tb + kk * CH, CH)])
      return 0
    nfull = RT // CH
    rem = RT - nfull * CH
    lax.fori_loop(0, nfull, zcp, 0)
    pltpu.sync_copy(wide0.at[pl.ds(0, rem)], ec_sh.at[pl.ds(tb + nfull * CH, rem)])

    pltpu.sync_copy(dst_hbm.at[wid], didx_v)
    plsc.subcore_barrier()

    ebase = wid * EPW
    def chunk(j, _):
      pltpu.async_copy(
          efa_hbm.at[pl.ds(ebase + j * CH, CH)], ebuf0, sla).wait()
      def wrow(i, _):
        wide0[i, pl.ds(0, 16)] = ebuf0[i, pl.ds(0, 16)]
        wide0[i, pl.ds(16, 16)] = ebuf0[i, pl.ds(16, 16)]
        return 0
      lax.fori_loop(0, CH, wrow, 0)
      pltpu.sync_copy(wide0, ec_sh.at[didx_v.at[j]], add=True)
      return 0
    lax.fori_loop(0, ITERS, chunk, 0)

    plsc.subcore_barrier()
    pltpu.sync_copy(ec_sh.at[pl.ds(tb, RT)], ec_out.at[c, pl.ds(tb, RT)])

  return k(efa, dst3)


def _tc_finish(g2, ec2, nfeats, wmsg_t, wapply_t, b_msg, b_apply):
  B = 2000
  grid = (N_NODES // B,)

  def body(g_ref, ec_ref, nf_ref, wm_ref, wa_ref, bm_ref, ba_ref, out_ref):
    g = g_ref[0] + g_ref[1]
    ec = ec_ref[0] + ec_ref[1]
    e = ec[:, :D_EDGE]
    cnt = ec[:, D_EDGE:D_EDGE + 1]
    summ = (
        jnp.dot(g, wm_ref[:D_IN], preferred_element_type=jnp.float32)
        + jnp.dot(e, wm_ref[D_IN:D_IN + D_EDGE], preferred_element_type=jnp.float32)
        + cnt * bm_ref[...]
    )
    h_neigh = summ / jnp.maximum(cnt, 1.0)
    ap = (
        jnp.dot(nf_ref[...], wa_ref[:D_IN], preferred_element_type=jnp.float32)
        + jnp.dot(h_neigh, wa_ref[D_IN:], preferred_element_type=jnp.float32)
        + ba_ref[...]
    )
    out_ref[...] = jnp.maximum(ap, 0.0)

  return pl.pallas_call(
      body,
      grid=grid,
      in_specs=[
          pl.BlockSpec((NC, B, D_IN), lambda i: (0, i, 0)),
          pl.BlockSpec((NC, B, D_IN), lambda i: (0, i, 0)),
          pl.BlockSpec((B, D_IN), lambda i: (i, 0)),
          pl.BlockSpec((D_IN + D_EDGE, D_OUT), lambda i: (0, 0)),
          pl.BlockSpec((D_IN + D_OUT, D_OUT), lambda i: (0, 0)),
          pl.BlockSpec((1, D_OUT), lambda i: (0, 0)),
          pl.BlockSpec((1, D_OUT), lambda i: (0, 0)),
      ],
      out_specs=pl.BlockSpec((B, D_OUT), lambda i: (i, 0)),
      out_shape=jax.ShapeDtypeStruct((N_NODES, D_OUT), jnp.float32),
  )(g2, ec2, nfeats, wmsg_t, wapply_t, b_msg, b_apply)


@jax.jit
def kernel(nfeats, efeats, edge_index, W_msg, b_msg, W_apply, b_apply):
  npad = EP - N_EDGES
  pad_ids = jnp.arange(npad, dtype=jnp.int32)
  src = jnp.concatenate(
      [edge_index[0].astype(jnp.int32), pad_ids % N_NODES])
  dst = jnp.concatenate(
      [edge_index[1].astype(jnp.int32),
       N_NODES + pad_ids % (R - N_NODES)])
  src = src.reshape(NW, ITERS, CH)
  dst = dst.reshape(NW, ITERS, CH)
  nfeats = nfeats.astype(jnp.float32)
  efa = jnp.concatenate(
      [jnp.concatenate(
          [efeats.astype(jnp.float32),
           jnp.ones((N_EDGES, 1), jnp.float32),
           jnp.zeros((N_EDGES, DEA - D_EDGE - 1), jnp.float32)], axis=1),
       jnp.zeros((npad, DEA), jnp.float32)])

  g2 = _sc_gather_nodes(nfeats, src, dst)
  ec2 = _sc_gather_edges(efa, dst)

  return _tc_finish(
      g2[:, :N_NODES], ec2[:, :N_NODES], nfeats,
      W_msg.T.astype(jnp.float32), W_apply.T.astype(jnp.float32),
      b_msg.reshape(1, D_OUT), b_apply.reshape(1, D_OUT))


# speedup vs baseline: 1.3935x; 1.0654x over previous
import functools

import jax
import jax.numpy as jnp
from jax import lax
from jax.experimental import pallas as pl
from jax.experimental.pallas import tpu as pltpu
from jax.experimental.pallas import tpu_sc as plsc

N_NODES = 10000
N_EDGES = 320000
D_IN = 128
D_EDGE = 16
D_OUT = 128

NC = 2
NS = 16
NW = NC * NS
CH = 128
R = 10112
RT = R // NS
ITERS = 79
EPW = ITERS * CH
EP = NW * EPW

_MESH = dict(core_axis_name="c", subcore_axis_name="s")


def _sc_gather_nodes(nfeats, src3, dst3):
  @functools.partial(
      pl.kernel,
      mesh=plsc.VectorSubcoreMesh(**_MESH),
      out_type=jax.ShapeDtypeStruct((NC, R, D_IN), jnp.float32),
      scratch_types=[
          pltpu.VMEM((CH, D_IN), jnp.float32),
          pltpu.VMEM((ITERS, CH), jnp.int32),
          pltpu.VMEM((ITERS, CH), jnp.int32),
          pltpu.VMEM_SHARED((R, D_IN), jnp.float32),
          pltpu.SemaphoreType.DMA,
      ],
  )
  def k(nf_hbm, src_hbm, dst_hbm, g_out, buf, sidx_v, didx_v, g_sh, sem):
    c = lax.axis_index("c")
    s = lax.axis_index("s")
    wid = c * NS + s
    tb = s * RT

    zeros16 = jnp.zeros((16,), jnp.float32)

    def zrow(i, _):
      for l in range(D_IN // 16):
        buf[i, pl.ds(l * 16, 16)] = zeros16
      return 0
    lax.fori_loop(0, CH, zrow, 0)

    def zcp(kk, _):
      pltpu.sync_copy(buf, g_sh.at[pl.ds(tb + kk * CH, CH)])
      return 0
    nfull = RT // CH
    rem = RT - nfull * CH
    lax.fori_loop(0, nfull, zcp, 0)
    pltpu.sync_copy(buf.at[pl.ds(0, rem)],
                    g_sh.at[pl.ds(tb + nfull * CH, rem)])

    pltpu.sync_copy(src_hbm.at[wid], sidx_v)
    pltpu.sync_copy(dst_hbm.at[wid], didx_v)
    plsc.subcore_barrier()

    def chunk(j, _):
      pltpu.async_copy(nf_hbm.at[sidx_v.at[j]], buf, sem).wait()
      pltpu.sync_copy(buf, g_sh.at[didx_v.at[j]], add=True)
      return 0
    lax.fori_loop(0, ITERS, chunk, 0)

    plsc.subcore_barrier()
    pltpu.sync_copy(g_sh.at[pl.ds(tb, RT)], g_out.at[c, pl.ds(tb, RT)])

  return k(nfeats, src3, dst3)


def _sc_gather_edges(ef, dst3):
  @functools.partial(
      pl.kernel,
      mesh=plsc.VectorSubcoreMesh(**_MESH),
      out_type=jax.ShapeDtypeStruct((NC, R, D_IN), jnp.float32),
      scratch_types=[
          pltpu.VMEM((CH, D_EDGE), jnp.float32),
          pltpu.VMEM((CH, D_IN), jnp.float32),
          pltpu.VMEM((ITERS, CH), jnp.int32),
          pltpu.VMEM_SHARED((R, D_IN), jnp.float32),
          pltpu.SemaphoreType.DMA,
      ],
  )
  def k(ef_hbm, dst_hbm, ec_out, ebuf, wide, didx_v, ec_sh, sem):
    c = lax.axis_index("c")
    s = lax.axis_index("s")
    wid = c * NS + s
    tb = s * RT

    zeros16 = jnp.zeros((16,), jnp.float32)
    def zrow(i, _):
      for l in range(D_IN // 16):
        wide[i, pl.ds(l * 16, 16)] = zeros16
      return 0
    lax.fori_loop(0, CH, zrow, 0)

    def zcp(kk, _):
      pltpu.sync_copy(wide, ec_sh.at[pl.ds(tb + kk * CH, CH)])
      return 0
    nfull = RT // CH
    rem = RT - nfull * CH
    lax.fori_loop(0, nfull, zcp, 0)
    pltpu.sync_copy(wide.at[pl.ds(0, rem)],
                    ec_sh.at[pl.ds(tb + nfull * CH, rem)])

    pltpu.sync_copy(dst_hbm.at[wid], didx_v)
    plsc.subcore_barrier()

    one0 = jnp.where(lax.iota(jnp.int32, 16) < 1, 1.0, 0.0)
    ebase = wid * EPW

    def chunk(j, _):
      pltpu.async_copy(
          ef_hbm.at[pl.ds(ebase + j * CH, CH)], ebuf, sem).wait()
      def wrow(i, _):
        wide[i, pl.ds(0, 16)] = ebuf[i, pl.ds(0, 16)]
        wide[i, pl.ds(16, 16)] = one0
        return 0
      lax.fori_loop(0, CH, wrow, 0)
      pltpu.sync_copy(wide, ec_sh.at[didx_v.at[j]], add=True)
      return 0
    lax.fori_loop(0, ITERS, chunk, 0)

    plsc.subcore_barrier()
    pltpu.sync_copy(ec_sh.at[pl.ds(tb, RT)], ec_out.at[c, pl.ds(tb, RT)])

  return k(ef, dst3)


def _tc_finish(g2, ec2, nfeats, wmsg_t, wapply_t, b_msg, b_apply):
  B = 2000
  grid = (N_NODES // B,)

  def body(g_ref, ec_ref, nf_ref, wm_ref, wa_ref, bm_ref, ba_ref, out_ref):
    g = g_ref[0] + g_ref[1]
    ec = ec_ref[0] + ec_ref[1]
    e = ec[:, :D_EDGE]
    cnt = ec[:, D_EDGE:D_EDGE + 1]
    summ = (
        jnp.dot(g, wm_ref[:D_IN], preferred_element_type=jnp.float32)
        + jnp.dot(e, wm_ref[D_IN:D_IN + D_EDGE], preferred_element_type=jnp.float32)
        + cnt * bm_ref[...]
    )
    h_neigh = summ / jnp.maximum(cnt, 1.0)
    ap = (
        jnp.dot(nf_ref[...], wa_ref[:D_IN], preferred_element_type=jnp.float32)
        + jnp.dot(h_neigh, wa_ref[D_IN:], preferred_element_type=jnp.float32)
        + ba_ref[...]
    )
    out_ref[...] = jnp.maximum(ap, 0.0)

  return pl.pallas_call(
      body,
      grid=grid,
      in_specs=[
          pl.BlockSpec((NC, B, D_IN), lambda i: (0, i, 0)),
          pl.BlockSpec((NC, B, D_IN), lambda i: (0, i, 0)),
          pl.BlockSpec((B, D_IN), lambda i: (i, 0)),
          pl.BlockSpec((D_IN + D_EDGE, D_OUT), lambda i: (0, 0)),
          pl.BlockSpec((D_IN + D_OUT, D_OUT), lambda i: (0, 0)),
          pl.BlockSpec((1, D_OUT), lambda i: (0, 0)),
          pl.BlockSpec((1, D_OUT), lambda i: (0, 0)),
      ],
      out_specs=pl.BlockSpec((B, D_OUT), lambda i: (i, 0)),
      out_shape=jax.ShapeDtypeStruct((N_NODES, D_OUT), jnp.float32),
  )(g2, ec2, nfeats, wmsg_t, wapply_t, b_msg, b_apply)


@jax.jit
def kernel(nfeats, efeats, edge_index, W_msg, b_msg, W_apply, b_apply):
  npad = EP - N_EDGES
  pad_ids = jnp.arange(npad, dtype=jnp.int32)
  src = jnp.concatenate(
      [edge_index[0].astype(jnp.int32), pad_ids % N_NODES])
  dst = jnp.concatenate(
      [edge_index[1].astype(jnp.int32),
       N_NODES + pad_ids % (R - N_NODES)])
  src = src.reshape(NW, ITERS, CH)
  dst = dst.reshape(NW, ITERS, CH)
  nfeats = nfeats.astype(jnp.float32)
  efp = jnp.concatenate(
      [efeats.astype(jnp.float32), jnp.zeros((npad, D_EDGE), jnp.float32)])

  g2 = _sc_gather_nodes(nfeats, src, dst)
  ec2 = _sc_gather_edges(efp, dst)

  return _tc_finish(
      g2, ec2, nfeats,
      W_msg.T.astype(jnp.float32), W_apply.T.astype(jnp.float32),
      b_msg.reshape(1, D_OUT), b_apply.reshape(1, D_OUT))


# speedup vs baseline: 1.4755x; 1.0588x over previous
import functools

import jax
import jax.numpy as jnp
from jax import lax
from jax.experimental import pallas as pl
from jax.experimental.pallas import tpu as pltpu
from jax.experimental.pallas import tpu_sc as plsc

N_NODES = 10000
N_EDGES = 320000
D_IN = 128
D_EDGE = 16
D_OUT = 128

NC = 2
NS = 16
NW = NC * NS
CH = 128
R = 10112
RT = R // NS
ITERS = 79
EPW = ITERS * CH
EP = NW * EPW

_MESH = dict(core_axis_name="c", subcore_axis_name="s")


def _sc_gather_nodes(nfeats, src3, dst3):
  @functools.partial(
      pl.kernel,
      mesh=plsc.VectorSubcoreMesh(**_MESH),
      out_type=jax.ShapeDtypeStruct((NC, R, D_IN), jnp.float32),
      scratch_types=[
          pltpu.VMEM((CH, D_IN), jnp.float32),
          pltpu.VMEM((ITERS, CH), jnp.int32),
          pltpu.VMEM((ITERS, CH), jnp.int32),
          pltpu.VMEM_SHARED((R, D_IN), jnp.float32),
          pltpu.SemaphoreType.DMA,
      ],
  )
  def k(nf_hbm, src_hbm, dst_hbm, g_out, buf, sidx_v, didx_v, g_sh, sem):
    c = lax.axis_index("c")
    s = lax.axis_index("s")
    wid = c * NS + s
    tb = s * RT

    zeros16 = jnp.zeros((16,), jnp.float32)

    def zrow(i, _):
      for l in range(D_IN // 16):
        buf[i, pl.ds(l * 16, 16)] = zeros16
      return 0
    lax.fori_loop(0, CH, zrow, 0)

    def zcp(kk, _):
      pltpu.sync_copy(buf, g_sh.at[pl.ds(tb + kk * CH, CH)])
      return 0
    nfull = RT // CH
    rem = RT - nfull * CH
    lax.fori_loop(0, nfull, zcp, 0)
    pltpu.sync_copy(buf.at[pl.ds(0, rem)],
                    g_sh.at[pl.ds(tb + nfull * CH, rem)])

    pltpu.sync_copy(src_hbm.at[wid], sidx_v)
    pltpu.sync_copy(dst_hbm.at[wid], didx_v)
    plsc.subcore_barrier()

    def chunk(j, _):
      pltpu.async_copy(nf_hbm.at[sidx_v.at[j]], buf, sem).wait()
      pltpu.sync_copy(buf, g_sh.at[didx_v.at[j]], add=True)
      return 0
    lax.fori_loop(0, ITERS, chunk, 0)

    plsc.subcore_barrier()
    pltpu.sync_copy(g_sh.at[pl.ds(tb, RT)], g_out.at[c, pl.ds(tb, RT)])

  return k(nfeats, src3, dst3)


def _sc_gather_edges(ef, dst3):
  @functools.partial(
      pl.kernel,
      mesh=plsc.VectorSubcoreMesh(**_MESH),
      out_type=jax.ShapeDtypeStruct((NC, R, D_IN), jnp.float32),
      scratch_types=[
          pltpu.VMEM((CH, D_EDGE), jnp.float32),
          pltpu.VMEM((CH, D_IN), jnp.float32),
          pltpu.VMEM((ITERS, CH), jnp.int32),
          pltpu.VMEM_SHARED((R, D_IN), jnp.float32),
          pltpu.SemaphoreType.DMA,
      ],
  )
  def k(ef_hbm, dst_hbm, ec_out, ebuf, wide, didx_v, ec_sh, sem):
    c = lax.axis_index("c")
    s = lax.axis_index("s")
    wid = c * NS + s
    tb = s * RT

    zeros16 = jnp.zeros((16,), jnp.float32)
    def zrow(i, _):
      for l in range(D_IN // 16):
        wide[i, pl.ds(l * 16, 16)] = zeros16
      return 0
    lax.fori_loop(0, CH, zrow, 0)

    def zcp(kk, _):
      pltpu.sync_copy(wide, ec_sh.at[pl.ds(tb + kk * CH, CH)])
      return 0
    nfull = RT // CH
    rem = RT - nfull * CH
    lax.fori_loop(0, nfull, zcp, 0)
    pltpu.sync_copy(wide.at[pl.ds(0, rem)],
                    ec_sh.at[pl.ds(tb + nfull * CH, rem)])

    pltpu.sync_copy(dst_hbm.at[wid], didx_v)
    plsc.subcore_barrier()

    one0 = jnp.where(lax.iota(jnp.int32, 16) < 1, 1.0, 0.0)
    ebase = wid * EPW

    def chunk(j, _):
      pltpu.async_copy(
          ef_hbm.at[pl.ds(ebase + j * CH, CH)], ebuf, sem).wait()
      def wrow(i4, _):
        for u in range(4):
          i = i4 * 4 + u
          wide[i, pl.ds(0, 16)] = ebuf[i, pl.ds(0, 16)]
          wide[i, pl.ds(16, 16)] = one0
        return 0
      lax.fori_loop(0, CH // 4, wrow, 0)
      pltpu.sync_copy(wide, ec_sh.at[didx_v.at[j]], add=True)
      return 0
    lax.fori_loop(0, ITERS, chunk, 0)

    plsc.subcore_barrier()
    pltpu.sync_copy(ec_sh.at[pl.ds(tb, RT)], ec_out.at[c, pl.ds(tb, RT)])

  return k(ef, dst3)


def _tc_finish(g2, ec2, nfeats, wmsg_t, wapply_t, b_msg, b_apply):
  B = 2000
  grid = (N_NODES // B,)

  def body(g_ref, ec_ref, nf_ref, wm_ref, wa_ref, bm_ref, ba_ref, out_ref):
    g = g_ref[0] + g_ref[1]
    ec = ec_ref[0] + ec_ref[1]
    e = ec[:, :D_EDGE]
    cnt = ec[:, D_EDGE:D_EDGE + 1]
    summ = (
        jnp.dot(g, wm_ref[:D_IN], preferred_element_type=jnp.float32)
        + jnp.dot(e, wm_ref[D_IN:D_IN + D_EDGE], preferred_element_type=jnp.float32)
        + cnt * bm_ref[...]
    )
    h_neigh = summ / jnp.maximum(cnt, 1.0)
    ap = (
        jnp.dot(nf_ref[...], wa_ref[:D_IN], preferred_element_type=jnp.float32)
        + jnp.dot(h_neigh, wa_ref[D_IN:], preferred_element_type=jnp.float32)
        + ba_ref[...]
    )
    out_ref[...] = jnp.maximum(ap, 0.0)

  return pl.pallas_call(
      body,
      grid=grid,
      in_specs=[
          pl.BlockSpec((NC, B, D_IN), lambda i: (0, i, 0)),
          pl.BlockSpec((NC, B, D_IN), lambda i: (0, i, 0)),
          pl.BlockSpec((B, D_IN), lambda i: (i, 0)),
          pl.BlockSpec((D_IN + D_EDGE, D_OUT), lambda i: (0, 0)),
          pl.BlockSpec((D_IN + D_OUT, D_OUT), lambda i: (0, 0)),
          pl.BlockSpec((1, D_OUT), lambda i: (0, 0)),
          pl.BlockSpec((1, D_OUT), lambda i: (0, 0)),
      ],
      out_specs=pl.BlockSpec((B, D_OUT), lambda i: (i, 0)),
      out_shape=jax.ShapeDtypeStruct((N_NODES, D_OUT), jnp.float32),
  )(g2, ec2, nfeats, wmsg_t, wapply_t, b_msg, b_apply)


@jax.jit
def kernel(nfeats, efeats, edge_index, W_msg, b_msg, W_apply, b_apply):
  npad = EP - N_EDGES
  pad_ids = jnp.arange(npad, dtype=jnp.int32)
  src = jnp.concatenate(
      [edge_index[0].astype(jnp.int32), pad_ids % N_NODES])
  dst = jnp.concatenate(
      [edge_index[1].astype(jnp.int32),
       N_NODES + pad_ids % (R - N_NODES)])
  src = src.reshape(NW, ITERS, CH)
  dst = dst.reshape(NW, ITERS, CH)
  nfeats = nfeats.astype(jnp.float32)
  efp = jnp.concatenate(
      [efeats.astype(jnp.float32), jnp.zeros((npad, D_EDGE), jnp.float32)])

  g2 = _sc_gather_nodes(nfeats, src, dst)
  ec2 = _sc_gather_edges(efp, dst)

  return _tc_finish(
      g2, ec2, nfeats,
      W_msg.T.astype(jnp.float32), W_apply.T.astype(jnp.float32),
      b_msg.reshape(1, D_OUT), b_apply.reshape(1, D_OUT))
